# baseline (speedup 1.0000x reference)
import functools

import jax
import jax.numpy as jnp
from jax import lax
from jax.experimental import pallas as pl
from jax.experimental.pallas import tpu as pltpu
from jax.experimental.pallas import tpu_sc as plsc

DIM = 64
NREL = 1000
BATCH = 16384
NC = 1
NS = 16
NW = NC * NS
ROWS_PER_W = BATCH // NW
L = 16
CH = 128
N_CHUNKS = ROWS_PER_W // CH
CH_GROUPS = CH // L


def _sc_body(subT_hbm, objT_hbm, rela_hbm, diagT_hbm, out_hbm,
             dg_sh, dg_v, rela_v, sub_v0, obj_v0, sub_v1, obj_v1,
             out_v, sem0, sem1):
    sid = lax.axis_index("s")
    wid = sid * NC + lax.axis_index("c")
    base = wid * ROWS_PER_W

    bufs = ((sub_v0, obj_v0, sem0), (sub_v1, obj_v1, sem1))

    def fire(k):
        sub_vb, obj_vb, semb = bufs[k % 2]
        cbase = base + k * CH
        return (
            pltpu.async_copy(subT_hbm.at[:, pl.ds(cbase, CH)], sub_vb, semb),
            pltpu.async_copy(objT_hbm.at[:, pl.ds(cbase, CH)], obj_vb, semb),
        )

    pending = [fire(0), fire(1)]
    pltpu.sync_copy(rela_hbm.at[pl.ds(base, ROWS_PER_W)], rela_v)

    @pl.when(sid == 0)
    def _():
        pltpu.sync_copy(diagT_hbm, dg_sh)

    plsc.subcore_barrier()
    pltpu.sync_copy(dg_sh, dg_v)

    def compute(k):
        sub_vb, obj_vb, _ = bufs[k % 2]

        def bgroup(bg, carry):
            off = k * CH + bg * L
            ridx = rela_v[pl.ds(off, L)]
            acc = None
            for d in range(DIM):
                s = sub_vb[d, pl.ds(bg * L, L)]
                o = obj_vb[d, pl.ds(bg * L, L)]
                r = plsc.load_gather(
                    dg_v, [jnp.full((L,), d, jnp.int32), ridx])
                p = s * o * r
                acc = p if acc is None else acc + p
            out_v[pl.ds(off, L)] = acc
            return carry

        lax.fori_loop(0, CH_GROUPS, bgroup, 0)

    for k in range(N_CHUNKS):
        for cp in pending.pop(0):
            cp.wait()
        compute(k)
        if k + 2 < N_CHUNKS:
            pending.append(fire(k + 2))

    pltpu.sync_copy(out_v, out_hbm.at[pl.ds(base, ROWS_PER_W)])


@functools.partial(
    pl.kernel,
    out_type=jax.ShapeDtypeStruct((BATCH,), jnp.float32),
    mesh=plsc.VectorSubcoreMesh(core_axis_name="c", subcore_axis_name="s",
                                num_cores=NC),
    compiler_params=pltpu.CompilerParams(needs_layout_passes=False,
                                         use_tc_tiling_on_sc=False),
    scratch_types=[
        pltpu.VMEM_SHARED((DIM, NREL), jnp.float32),
        pltpu.VMEM((DIM, NREL), jnp.float32),
        pltpu.VMEM((ROWS_PER_W,), jnp.int32),
        pltpu.VMEM((DIM, CH), jnp.float32),
        pltpu.VMEM((DIM, CH), jnp.float32),
        pltpu.VMEM((DIM, CH), jnp.float32),
        pltpu.VMEM((DIM, CH), jnp.float32),
        pltpu.VMEM((ROWS_PER_W,), jnp.float32),
        pltpu.SemaphoreType.DMA,
        pltpu.SemaphoreType.DMA,
    ],
)
def _dist_mult_sc(subT_hbm, objT_hbm, rela_hbm, diagT_hbm, out_hbm, *scratch):
    _sc_body(subT_hbm, objT_hbm, rela_hbm, diagT_hbm, out_hbm, *scratch)


def kernel(sub_embed, obj_embed, rela, diag):
    return _dist_mult_sc(sub_embed.T, obj_embed.T,
                         rela.astype(jnp.int32), diag.T)


# speedup vs baseline: 1.0169x; 1.0169x over previous
import functools

import jax
import jax.numpy as jnp
from jax import lax
from jax.experimental import pallas as pl
from jax.experimental.pallas import tpu as pltpu
from jax.experimental.pallas import tpu_sc as plsc

DIM = 64
NREL = 1000
BATCH = 16384
NC = 2
NS = 16
NW = NC * NS
ROWS_PER_W = BATCH // NW
L = 16
CH = 128
N_CHUNKS = ROWS_PER_W // CH
CH_GROUPS = CH // L


def _sc_body(subT_hbm, objT_hbm, rela_hbm, diagT_hbm, out_hbm,
             dg_v, rela_v, sub_v0, obj_v0, sub_v1, obj_v1,
             out_v, sem0, sem1):
    wid = lax.axis_index("s") * NC + lax.axis_index("c")
    base = wid * ROWS_PER_W

    bufs = ((sub_v0, obj_v0, sem0), (sub_v1, obj_v1, sem1))

    def fire(k):
        sub_vb, obj_vb, semb = bufs[k % 2]
        cbase = base + k * CH
        return (
            pltpu.async_copy(subT_hbm.at[:, pl.ds(cbase, CH)], sub_vb, semb),
            pltpu.async_copy(objT_hbm.at[:, pl.ds(cbase, CH)], obj_vb, semb),
        )

    pending = [fire(0), fire(1)]
    pltpu.sync_copy(rela_hbm.at[pl.ds(base, ROWS_PER_W)], rela_v)

    pltpu.sync_copy(diagT_hbm, dg_v)

    def compute(k):
        sub_vb, obj_vb, _ = bufs[k % 2]

        def bgroup(bg, carry):
            off = k * CH + bg * L
            ridx = rela_v[pl.ds(off, L)]
            acc = None
            for d in range(DIM):
                s = sub_vb[d, pl.ds(bg * L, L)]
                o = obj_vb[d, pl.ds(bg * L, L)]
                r = plsc.load_gather(
                    dg_v, [jnp.full((L,), d, jnp.int32), ridx])
                p = s * o * r
                acc = p if acc is None else acc + p
            out_v[pl.ds(off, L)] = acc
            return carry

        lax.fori_loop(0, CH_GROUPS, bgroup, 0)

    for k in range(N_CHUNKS):
        for cp in pending.pop(0):
            cp.wait()
        compute(k)
        if k + 2 < N_CHUNKS:
            pending.append(fire(k + 2))

    pltpu.sync_copy(out_v, out_hbm.at[pl.ds(base, ROWS_PER_W)])


@functools.partial(
    pl.kernel,
    out_type=jax.ShapeDtypeStruct((BATCH,), jnp.float32),
    mesh=plsc.VectorSubcoreMesh(core_axis_name="c", subcore_axis_name="s"),
    compiler_params=pltpu.CompilerParams(needs_layout_passes=False,
                                         use_tc_tiling_on_sc=False),
    scratch_types=[
        pltpu.VMEM((DIM, NREL), jnp.float32),
        pltpu.VMEM((ROWS_PER_W,), jnp.int32),
        pltpu.VMEM((DIM, CH), jnp.float32),
        pltpu.VMEM((DIM, CH), jnp.float32),
        pltpu.VMEM((DIM, CH), jnp.float32),
        pltpu.VMEM((DIM, CH), jnp.float32),
        pltpu.VMEM((ROWS_PER_W,), jnp.float32),
        pltpu.SemaphoreType.DMA,
        pltpu.SemaphoreType.DMA,
    ],
)
def _dist_mult_sc(subT_hbm, objT_hbm, rela_hbm, diagT_hbm, out_hbm, *scratch):
    _sc_body(subT_hbm, objT_hbm, rela_hbm, diagT_hbm, out_hbm, *scratch)


def kernel(sub_embed, obj_embed, rela, diag):
    return _dist_mult_sc(sub_embed.T, obj_embed.T,
                         rela.astype(jnp.int32), diag.T)


# speedup vs baseline: 1.1342x; 1.1154x over previous
import functools

import jax
import jax.numpy as jnp
from jax import lax
from jax.experimental import pallas as pl
from jax.experimental.pallas import tpu as pltpu
from jax.experimental.pallas import tpu_sc as plsc

DIM = 64
NREL = 1000
BATCH = 16384
NC = 2
NS = 16
NW = NC * NS
ROWS_PER_W = BATCH // NW
L = 16
CH = 128
N_CHUNKS = ROWS_PER_W // CH
CH_GROUPS = CH // L


def _sc_body(subT_hbm, objT_hbm, rela_hbm, diagT_hbm, out_hbm,
             dg_sh, dg_v, rela_v, sub_v0, obj_v0, sub_v1, obj_v1,
             out_v, sem0, sem1):
    sid = lax.axis_index("s")
    wid = sid * NC + lax.axis_index("c")
    base = wid * ROWS_PER_W

    bufs = ((sub_v0, obj_v0, sem0), (sub_v1, obj_v1, sem1))

    def fire(k):
        sub_vb, obj_vb, semb = bufs[k % 2]
        cbase = base + k * CH
        return (
            pltpu.async_copy(subT_hbm.at[:, pl.ds(cbase, CH)], sub_vb, semb),
            pltpu.async_copy(objT_hbm.at[:, pl.ds(cbase, CH)], obj_vb, semb),
        )

    pending = [fire(0), fire(1)]
    pltpu.sync_copy(rela_hbm.at[pl.ds(base, ROWS_PER_W)], rela_v)

    @pl.when(sid == 0)
    def _():
        pltpu.sync_copy(diagT_hbm, dg_sh)

    plsc.subcore_barrier()
    pltpu.sync_copy(dg_sh, dg_v)

    def compute(k):
        sub_vb, obj_vb, _ = bufs[k % 2]

        def bgroup(bg, carry):
            off = k * CH + bg * L
            ridx = rela_v[pl.ds(off, L)]
            acc = None
            for d in range(DIM):
                s = sub_vb[d, pl.ds(bg * L, L)]
                o = obj_vb[d, pl.ds(bg * L, L)]
                r = plsc.load_gather(dg_v.at[d], [ridx])
                p = s * o * r
                acc = p if acc is None else acc + p
            out_v[pl.ds(off, L)] = acc
            return carry

        lax.fori_loop(0, CH_GROUPS, bgroup, 0)

    for k in range(N_CHUNKS):
        for cp in pending.pop(0):
            cp.wait()
        compute(k)
        if k + 2 < N_CHUNKS:
            pending.append(fire(k + 2))

    pltpu.sync_copy(out_v, out_hbm.at[pl.ds(base, ROWS_PER_W)])


@functools.partial(
    pl.kernel,
    out_type=jax.ShapeDtypeStruct((BATCH,), jnp.float32),
    mesh=plsc.VectorSubcoreMesh(core_axis_name="c", subcore_axis_name="s"),
    compiler_params=pltpu.CompilerParams(needs_layout_passes=False,
                                         use_tc_tiling_on_sc=False),
    scratch_types=[
        pltpu.VMEM_SHARED((DIM, NREL), jnp.float32),
        pltpu.VMEM((DIM, NREL), jnp.float32),
        pltpu.VMEM((ROWS_PER_W,), jnp.int32),
        pltpu.VMEM((DIM, CH), jnp.float32),
        pltpu.VMEM((DIM, CH), jnp.float32),
        pltpu.VMEM((DIM, CH), jnp.float32),
        pltpu.VMEM((DIM, CH), jnp.float32),
        pltpu.VMEM((ROWS_PER_W,), jnp.float32),
        pltpu.SemaphoreType.DMA,
        pltpu.SemaphoreType.DMA,
    ],
)
def _dist_mult_sc(subT_hbm, objT_hbm, rela_hbm, diagT_hbm, out_hbm, *scratch):
    _sc_body(subT_hbm, objT_hbm, rela_hbm, diagT_hbm, out_hbm, *scratch)


def kernel(sub_embed, obj_embed, rela, diag):
    return _dist_mult_sc(sub_embed.T, obj_embed.T,
                         rela.astype(jnp.int32), diag.T)
